# baseline (speedup 1.0000x reference)
import jax
import jax.numpy as jnp
from jax.experimental import pallas as pl
from jax.experimental.pallas import tpu as pltpu

_P = 16
_H = 14
_L = _H * _H
_K = _P * _P * 3
_B = 8


def _body(imgs_ref, pred_ref, mask_ref, out_ref, acc_ref, s_ref):
    i = pl.program_id(0)
    n = pl.num_programs(0)

    @pl.when(i == 0)
    def _init():
        acc_ref[0] = 0.0
        acc_ref[1] = 0.0
        k_i = jax.lax.broadcasted_iota(jnp.int32, (_K, _K), 0)
        j_i = jax.lax.broadcasted_iota(jnp.int32, (_K, _K), 1)
        cond = k_i == 3 * (j_i % 256) + j_i // 256
        s_ref[...] = jnp.where(cond, 1.0, 0.0).astype(jnp.bfloat16)

    def sample(j, carry):
        num, den = carry
        x4 = imgs_ref[j].astype(jnp.bfloat16)
        x6 = x4.reshape(3, _H, _P, _H, _P)
        xt = jnp.transpose(x6, (1, 3, 0, 2, 4))
        t = xt.reshape(_L, _K)

        sx = jnp.sum(t, axis=-1, keepdims=True, dtype=jnp.float32)
        sxx = jnp.sum(t * t, axis=-1, keepdims=True, dtype=jnp.float32)
        mean = sx * (1.0 / _K)
        var = (sxx - sx * mean) * (1.0 / (_K - 1))
        rstd = jax.lax.rsqrt(var + 1e-6)

        ps = jnp.dot(pred_ref[j].astype(jnp.bfloat16), s_ref[...],
                     preferred_element_type=jnp.float32).astype(jnp.bfloat16)

        tn = (t - mean.astype(jnp.bfloat16)) * rstd.astype(jnp.bfloat16)
        e = ps - tn
        lp = jnp.sum(e * e, axis=-1, dtype=jnp.float32) * (1.0 / _K)
        m = mask_ref[0, j]
        return (num + jnp.sum(lp * m), den + jnp.sum(m))

    num, den = jax.lax.fori_loop(0, _B, sample, (0.0, 0.0))
    acc_ref[0] += num
    acc_ref[1] += den

    @pl.when(i == n - 1)
    def _fin():
        out_ref[...] = jnp.full((1, 1), acc_ref[0] / acc_ref[1], jnp.float32)


def kernel(imgs, pred, mask):
    N = imgs.shape[0]
    imgs5 = imgs.reshape(N, 3, _H, _P, _H * _P)
    out = pl.pallas_call(
        _body,
        grid=(N // _B,),
        in_specs=[
            pl.BlockSpec((_B, 3, _H, _P, _H * _P), lambda i: (i, 0, 0, 0, 0)),
            pl.BlockSpec((_B, _L, _K), lambda i: (i, 0, 0)),
            pl.BlockSpec((1, _B, _L), lambda i: (i, 0, 0)),
        ],
        out_specs=pl.BlockSpec((1, 1), lambda i: (0, 0)),
        out_shape=jax.ShapeDtypeStruct((1, 1), jnp.float32),
        scratch_shapes=[pltpu.SMEM((2,), jnp.float32),
                        pltpu.VMEM((_K, _K), jnp.bfloat16)],
    )(imgs5, pred, mask.reshape(N // _B, _B, _L))
    return out[0, 0]


# speedup vs baseline: 1.0323x; 1.0323x over previous
import jax
import jax.numpy as jnp
from jax.experimental import pallas as pl
from jax.experimental.pallas import tpu as pltpu

_P = 16
_H = 14
_L = _H * _H
_K = _P * _P * 3
_B = 8
_V = 2


def _body(imgs_ref, pred_ref, mask_ref, out_ref, acc_ref, s_ref):
    i = pl.program_id(0)
    n = pl.num_programs(0)

    @pl.when(i == 0)
    def _init():
        acc_ref[0] = 0.0
        acc_ref[1] = 0.0
        k_i = jax.lax.broadcasted_iota(jnp.int32, (_K, _K), 0)
        j_i = jax.lax.broadcasted_iota(jnp.int32, (_K, _K), 1)
        cond = k_i == 3 * (j_i % 256) + j_i // 256
        s_ref[...] = jnp.where(cond, 1.0, 0.0).astype(jnp.bfloat16)

    def pair(j, carry):
        num, den = carry
        x4 = imgs_ref[pl.ds(_V * j, _V)].astype(jnp.bfloat16)
        x6 = x4.reshape(_V, 3, _H, _P, _H, _P)
        xt = jnp.transpose(x6, (0, 2, 4, 1, 3, 5))
        t = xt.reshape(_V * _L, _K)

        sx = jnp.sum(t, axis=-1, keepdims=True, dtype=jnp.float32)
        sxx = jnp.sum(t * t, axis=-1, keepdims=True, dtype=jnp.float32)
        mean = sx * (1.0 / _K)
        var = (sxx - sx * mean) * (1.0 / (_K - 1))
        rstd = jax.lax.rsqrt(var + 1e-6)

        ps = jnp.dot(
            pred_ref[pl.ds(_V * j, _V)].reshape(_V * _L, _K).astype(jnp.bfloat16),
            s_ref[...], preferred_element_type=jnp.float32).astype(jnp.bfloat16)

        tn = (t - mean.astype(jnp.bfloat16)) * rstd.astype(jnp.bfloat16)
        e = ps - tn
        lp = jnp.sum(e * e, axis=-1, dtype=jnp.float32) * (1.0 / _K)
        m = mask_ref[0, j]
        return (num + jnp.sum(lp * m), den + jnp.sum(m))

    num, den = jax.lax.fori_loop(0, _B // _V, pair, (0.0, 0.0))
    acc_ref[0] += num
    acc_ref[1] += den

    @pl.when(i == n - 1)
    def _fin():
        out_ref[...] = jnp.full((1, 1), acc_ref[0] / acc_ref[1], jnp.float32)


def kernel(imgs, pred, mask):
    N = imgs.shape[0]
    imgs5 = imgs.reshape(N, 3, _H, _P, _H * _P)
    out = pl.pallas_call(
        _body,
        grid=(N // _B,),
        in_specs=[
            pl.BlockSpec((_B, 3, _H, _P, _H * _P), lambda i: (i, 0, 0, 0, 0)),
            pl.BlockSpec((_B, _L, _K), lambda i: (i, 0, 0)),
            pl.BlockSpec((1, _B // _V, _V * _L), lambda i: (i, 0, 0)),
        ],
        out_specs=pl.BlockSpec((1, 1), lambda i: (0, 0)),
        out_shape=jax.ShapeDtypeStruct((1, 1), jnp.float32),
        scratch_shapes=[pltpu.SMEM((2,), jnp.float32),
                        pltpu.VMEM((_K, _K), jnp.bfloat16)],
    )(imgs5, pred, mask.reshape(N // _B, _B // _V, _V * _L))
    return out[0, 0]
